# baseline (speedup 1.0000x reference)
import jax
import jax.numpy as jnp
from jax import lax
from jax.experimental import pallas as pl
from jax.experimental.pallas import tpu as pltpu
from jax.experimental.pallas import tpu_sc as plsc

NC = 2
NS = 16
L = 16
NW = NC * NS

B = 16384
D = 32
BPW = B // NW
CH = 128
KCH = BPW // CH
GROUPS = BPW // L


def _gmf_body(uidx_hbm, iidx_hbm, utab_hbm, itab_hbm, w_hbm, b_hbm, out_hbm,
              uidx_v, iidx_v, urows_v, irows_v, w_v, b_v, out_v,
              sem_u, sem_i):
    wid = lax.axis_index("s") * NC + lax.axis_index("c")

    pltpu.sync_copy(uidx_hbm.at[pl.ds(wid * KCH, KCH)], uidx_v)
    pltpu.sync_copy(iidx_hbm.at[pl.ds(wid * KCH, KCH)], iidx_v)

    copies = []
    for j in range(KCH):
        copies.append(pltpu.async_copy(
            utab_hbm.at[uidx_v.at[j]], urows_v.at[pl.ds(j * CH, CH)], sem_u))
        copies.append(pltpu.async_copy(
            itab_hbm.at[iidx_v.at[j]], irows_v.at[pl.ds(j * CH, CH)], sem_i))
    pltpu.sync_copy(w_hbm, w_v)
    pltpu.sync_copy(b_hbm, b_v)
    for c in copies:
        c.wait()

    bias = b_v[...][0]
    w_lo = w_v[pl.ds(0, L)]
    w_hi = w_v[pl.ds(L, L)]

    lane = lax.iota(jnp.int32, L)

    def group(g, carry):
        base = g * L
        acc = jnp.zeros((L,), jnp.float32)
        for r in range(L):
            u0 = urows_v[base + r, pl.ds(0, L)]
            u1 = urows_v[base + r, pl.ds(L, L)]
            i0 = irows_v[base + r, pl.ds(0, L)]
            i1 = irows_v[base + r, pl.ds(L, L)]
            p = (u0 * i0) * w_lo + (u1 * i1) * w_hi
            s = jnp.sum(p)
            acc = jnp.where(lane == r, s, acc)
        logits = acc + bias
        rating = 1.0 / (1.0 + jnp.exp(-logits))
        out_v[pl.ds(g * L, L)] = rating
        return carry

    lax.fori_loop(0, GROUPS, group, 0)
    pltpu.sync_copy(out_v, out_hbm.at[pl.ds(wid * BPW, BPW)])


@jax.jit
def _gmf(uidx, iidx, utab, itab, w, bias):
    mesh = plsc.VectorSubcoreMesh(core_axis_name="c", subcore_axis_name="s")
    kfn = pl.kernel(
        _gmf_body,
        out_type=jax.ShapeDtypeStruct((B,), jnp.float32),
        mesh=mesh,
        compiler_params=pltpu.CompilerParams(
            needs_layout_passes=False, use_tc_tiling_on_sc=False),
        scratch_types=[
            pltpu.VMEM((KCH, CH), jnp.int32),
            pltpu.VMEM((KCH, CH), jnp.int32),
            pltpu.VMEM((BPW, D), jnp.float32),
            pltpu.VMEM((BPW, D), jnp.float32),
            pltpu.VMEM((D,), jnp.float32),
            pltpu.VMEM((L,), jnp.float32),
            pltpu.VMEM((BPW,), jnp.float32),
            pltpu.SemaphoreType.DMA,
            pltpu.SemaphoreType.DMA,
        ],
    )
    return kfn(uidx, iidx, utab, itab, w, bias)


def kernel(user_indices, item_indices, user_table, item_table, W, b):
    uidx = user_indices.astype(jnp.int32).reshape(B // CH, CH)
    iidx = item_indices.astype(jnp.int32).reshape(B // CH, CH)
    w = W.astype(jnp.float32).reshape(D)
    bias = jnp.pad(b.astype(jnp.float32).reshape(1), (0, L - 1))
    out = _gmf(uidx, iidx, user_table, item_table, w, bias)
    return out.reshape(B, 1)


# speedup vs baseline: 3.3474x; 3.3474x over previous
import jax
import jax.numpy as jnp
from jax import lax
from jax.experimental import pallas as pl
from jax.experimental.pallas import tpu as pltpu
from jax.experimental.pallas import tpu_sc as plsc

NC = 2
NS = 16
L = 16
NW = NC * NS

B = 16384
D = 32
N = 1000000
BPW = B // NW
GROUPS = BPW // L
SLOTS = 8


def _gmf_body(uidx_hbm, iidx_hbm, utab_hbm, itab_hbm, w_hbm, b_hbm, out_hbm,
              uidx_v, iidx_v, ustage_v, istage_v, w_v, b_v, out_v,
              sem_u, sem_i):
    wid = lax.axis_index("s") * NC + lax.axis_index("c")
    base = wid * BPW

    pltpu.sync_copy(uidx_hbm.at[pl.ds(base, BPW)], uidx_v)
    pltpu.sync_copy(iidx_hbm.at[pl.ds(base, BPW)], iidx_v)
    pltpu.sync_copy(w_hbm, w_v)
    pltpu.sync_copy(b_hbm, b_v)

    bias = b_v[...][0]
    w_lo = w_v[pl.ds(0, L)]
    w_hi = w_v[pl.ds(L, L)]
    lane = lax.iota(jnp.int32, L)
    c_lo = lane
    c_hi = lane + L

    def group(g, _):
        ru = uidx_v[pl.ds(g * L, L)]
        ri = iidx_v[pl.ds(g * L, L)]
        acc = jnp.zeros((L,), jnp.float32)
        for h in range(L // SLOTS):
            copies = []
            for k in range(SLOTS):
                r_u = ru[h * SLOTS + k]
                r_i = ri[h * SLOTS + k]
                cb_u = pl.multiple_of((r_u >> 7) * 128, 128)
                cb_i = pl.multiple_of((r_i >> 7) * 128, 128)
                copies.append(pltpu.async_copy(
                    utab_hbm.at[:, pl.ds(cb_u, 128)], ustage_v.at[k], sem_u))
                copies.append(pltpu.async_copy(
                    itab_hbm.at[:, pl.ds(cb_i, 128)], istage_v.at[k], sem_i))
            for c in copies:
                c.wait()
            for k in range(SLOTS):
                rl_u = jnp.full((L,), ru[h * SLOTS + k] & 127, jnp.int32)
                rl_i = jnp.full((L,), ri[h * SLOTS + k] & 127, jnp.int32)
                slot = jnp.full((L,), k, jnp.int32)
                u0 = plsc.load_gather(ustage_v, [slot, c_lo, rl_u])
                u1 = plsc.load_gather(ustage_v, [slot, c_hi, rl_u])
                i0 = plsc.load_gather(istage_v, [slot, c_lo, rl_i])
                i1 = plsc.load_gather(istage_v, [slot, c_hi, rl_i])
                s = jnp.sum((u0 * i0) * w_lo + (u1 * i1) * w_hi)
                acc = jnp.where(lane == h * SLOTS + k, s, acc)
        logits = acc + bias
        rating = 1.0 / (1.0 + jnp.exp(-logits))
        out_v[pl.ds(g * L, L)] = rating
        return 0

    lax.fori_loop(0, GROUPS, group, 0)
    pltpu.sync_copy(out_v, out_hbm.at[pl.ds(base, BPW)])


@jax.jit
def _gmf(uidx, iidx, utab_t, itab_t, w, bias):
    mesh = plsc.VectorSubcoreMesh(core_axis_name="c", subcore_axis_name="s")
    kfn = pl.kernel(
        _gmf_body,
        out_type=jax.ShapeDtypeStruct((B,), jnp.float32),
        mesh=mesh,
        compiler_params=pltpu.CompilerParams(
            needs_layout_passes=False, use_tc_tiling_on_sc=True),
        scratch_types=[
            pltpu.VMEM((BPW,), jnp.int32),
            pltpu.VMEM((BPW,), jnp.int32),
            pltpu.VMEM((SLOTS, D, 128), jnp.float32),
            pltpu.VMEM((SLOTS, D, 128), jnp.float32),
            pltpu.VMEM((D,), jnp.float32),
            pltpu.VMEM((L,), jnp.float32),
            pltpu.VMEM((BPW,), jnp.float32),
            pltpu.SemaphoreType.DMA,
            pltpu.SemaphoreType.DMA,
        ],
    )
    return kfn(uidx, iidx, utab_t, itab_t, w, bias)


def kernel(user_indices, item_indices, user_table, item_table, W, b):
    uidx = user_indices.astype(jnp.int32)
    iidx = item_indices.astype(jnp.int32)
    w = W.astype(jnp.float32).reshape(D)
    bias = jnp.pad(b.astype(jnp.float32).reshape(1), (0, L - 1))
    out = _gmf(uidx, iidx, user_table.T, item_table.T, w, bias)
    return out.reshape(B, 1)


# speedup vs baseline: 3.7185x; 1.1108x over previous
import jax
import jax.numpy as jnp
from jax import lax
from jax.experimental import pallas as pl
from jax.experimental.pallas import tpu as pltpu
from jax.experimental.pallas import tpu_sc as plsc

NC = 2
NS = 16
L = 16
NW = NC * NS

B = 16384
D = 32
N = 1000000
BPW = B // NW
GROUPS = BPW // L
SLOTS = 4


def _gmf_body(uidx_hbm, iidx_hbm, utab_hbm, itab_hbm, w_hbm, b_hbm, out_hbm,
              uidx_v, iidx_v, ustage_v, istage_v, w_v, b_v, out_v,
              sem_u, sem_i):
    wid = lax.axis_index("s") * NC + lax.axis_index("c")
    base = wid * BPW

    pltpu.sync_copy(uidx_hbm.at[pl.ds(base, BPW)], uidx_v)
    pltpu.sync_copy(iidx_hbm.at[pl.ds(base, BPW)], iidx_v)
    pltpu.sync_copy(w_hbm, w_v)
    pltpu.sync_copy(b_hbm, b_v)

    bias = b_v[...][0]
    w_lo = w_v[pl.ds(0, L)]
    w_hi = w_v[pl.ds(L, L)]
    lane = lax.iota(jnp.int32, L)
    c_lo = lane
    c_hi = lane + L

    def group(g, _):
        ru = uidx_v[pl.ds(g * L, L)]
        ri = iidx_v[pl.ds(g * L, L)]

        pending = [[], []]

        def fetch(q):
            st = q % 2
            for k in range(SLOTS):
                r_u = ru[q * SLOTS + k]
                r_i = ri[q * SLOTS + k]
                cb_u = pl.multiple_of((r_u >> 7) * 128, 128)
                cb_i = pl.multiple_of((r_i >> 7) * 128, 128)
                pending[st].append(pltpu.async_copy(
                    utab_hbm.at[:, pl.ds(cb_u, 128)], ustage_v.at[st, k],
                    sem_u))
                pending[st].append(pltpu.async_copy(
                    itab_hbm.at[:, pl.ds(cb_i, 128)], istage_v.at[st, k],
                    sem_i))

        fetch(0)
        acc = jnp.zeros((L,), jnp.float32)
        for q in range(L // SLOTS):
            st = q % 2
            if q + 1 < L // SLOTS:
                fetch(q + 1)
            for c in pending[st]:
                c.wait()
            pending[st] = []
            for k in range(SLOTS):
                rl_u = jnp.full((L,), ru[q * SLOTS + k] & 127, jnp.int32)
                rl_i = jnp.full((L,), ri[q * SLOTS + k] & 127, jnp.int32)
                sv = jnp.full((L,), st, jnp.int32)
                slot = jnp.full((L,), k, jnp.int32)
                u0 = plsc.load_gather(ustage_v, [sv, slot, c_lo, rl_u])
                u1 = plsc.load_gather(ustage_v, [sv, slot, c_hi, rl_u])
                i0 = plsc.load_gather(istage_v, [sv, slot, c_lo, rl_i])
                i1 = plsc.load_gather(istage_v, [sv, slot, c_hi, rl_i])
                s = jnp.sum((u0 * i0) * w_lo + (u1 * i1) * w_hi)
                acc = jnp.where(lane == q * SLOTS + k, s, acc)
        logits = acc + bias
        rating = 1.0 / (1.0 + jnp.exp(-logits))
        out_v[pl.ds(g * L, L)] = rating
        return 0

    lax.fori_loop(0, GROUPS, group, 0)
    pltpu.sync_copy(out_v, out_hbm.at[pl.ds(base, BPW)])


@jax.jit
def _gmf(uidx, iidx, utab_t, itab_t, w, bias):
    mesh = plsc.VectorSubcoreMesh(core_axis_name="c", subcore_axis_name="s")
    kfn = pl.kernel(
        _gmf_body,
        out_type=jax.ShapeDtypeStruct((B,), jnp.float32),
        mesh=mesh,
        compiler_params=pltpu.CompilerParams(
            needs_layout_passes=False, use_tc_tiling_on_sc=True),
        scratch_types=[
            pltpu.VMEM((BPW,), jnp.int32),
            pltpu.VMEM((BPW,), jnp.int32),
            pltpu.VMEM((2, SLOTS, D, 128), jnp.float32),
            pltpu.VMEM((2, SLOTS, D, 128), jnp.float32),
            pltpu.VMEM((D,), jnp.float32),
            pltpu.VMEM((L,), jnp.float32),
            pltpu.VMEM((BPW,), jnp.float32),
            pltpu.SemaphoreType.DMA,
            pltpu.SemaphoreType.DMA,
        ],
    )
    return kfn(uidx, iidx, utab_t, itab_t, w, bias)


def kernel(user_indices, item_indices, user_table, item_table, W, b):
    uidx = user_indices.astype(jnp.int32)
    iidx = item_indices.astype(jnp.int32)
    w = W.astype(jnp.float32).reshape(D)
    bias = jnp.pad(b.astype(jnp.float32).reshape(1), (0, L - 1))
    out = _gmf(uidx, iidx, user_table.T, item_table.T, w, bias)
    return out.reshape(B, 1)


# speedup vs baseline: 3.9774x; 1.0696x over previous
import jax
import jax.numpy as jnp
from jax import lax
from jax.experimental import pallas as pl
from jax.experimental.pallas import tpu as pltpu
from jax.experimental.pallas import tpu_sc as plsc

NC = 2
NS = 16
L = 16
NW = NC * NS

B = 16384
D = 32
N = 1000000
BPW = B // NW
GROUPS = BPW // L
SLOTS = 4


def _gmf_body(uidx_hbm, iidx_hbm, utab_hbm, itab_hbm, w_hbm, b_hbm, out_hbm,
              uidx_v, iidx_v, ustage_v, istage_v, w_v, b_v, out_v,
              sem_u, sem_i):
    wid = lax.axis_index("s") * NC + lax.axis_index("c")
    base = wid * BPW

    pltpu.sync_copy(uidx_hbm.at[pl.ds(base, BPW)], uidx_v)
    pltpu.sync_copy(iidx_hbm.at[pl.ds(base, BPW)], iidx_v)
    pltpu.sync_copy(w_hbm, w_v)
    pltpu.sync_copy(b_hbm, b_v)

    bias = b_v[...][0]
    w_lo = w_v[pl.ds(0, L)]
    w_hi = w_v[pl.ds(L, L)]
    lane = lax.iota(jnp.int32, L)
    c_lo = lane
    c_hi = lane + L

    NQ = (2 * L) // SLOTS

    def group(g, _):
        ru = [uidx_v[pl.ds(g * 2 * L, L)], uidx_v[pl.ds(g * 2 * L + L, L)]]
        ri = [iidx_v[pl.ds(g * 2 * L, L)], iidx_v[pl.ds(g * 2 * L + L, L)]]

        pending = [[], [], []]

        def fetch(q):
            st = q % 3
            h = q // (L // SLOTS)
            for k in range(SLOTS):
                j = (q % (L // SLOTS)) * SLOTS + k
                cb_u = pl.multiple_of((ru[h][j] >> 7) * 128, 128)
                cb_i = pl.multiple_of((ri[h][j] >> 7) * 128, 128)
                pending[st].append(pltpu.async_copy(
                    utab_hbm.at[:, pl.ds(cb_u, 128)], ustage_v.at[st, k],
                    sem_u))
                pending[st].append(pltpu.async_copy(
                    itab_hbm.at[:, pl.ds(cb_i, 128)], istage_v.at[st, k],
                    sem_i))

        fetch(0)
        fetch(1)
        acc = [jnp.zeros((L,), jnp.float32), jnp.zeros((L,), jnp.float32)]
        for q in range(NQ):
            st = q % 3
            h = q // (L // SLOTS)
            if q + 2 < NQ:
                fetch(q + 2)
            for c in pending[st]:
                c.wait()
            pending[st] = []
            for k in range(SLOTS):
                j = (q % (L // SLOTS)) * SLOTS + k
                rl_u = jnp.full((L,), ru[h][j] & 127, jnp.int32)
                rl_i = jnp.full((L,), ri[h][j] & 127, jnp.int32)
                sv = jnp.full((L,), st, jnp.int32)
                slot = jnp.full((L,), k, jnp.int32)
                u0 = plsc.load_gather(ustage_v, [sv, slot, c_lo, rl_u])
                u1 = plsc.load_gather(ustage_v, [sv, slot, c_hi, rl_u])
                i0 = plsc.load_gather(istage_v, [sv, slot, c_lo, rl_i])
                i1 = plsc.load_gather(istage_v, [sv, slot, c_hi, rl_i])
                s = jnp.sum((u0 * i0) * w_lo + (u1 * i1) * w_hi)
                acc[h] = jnp.where(lane == j, s, acc[h])
        for h in range(2):
            logits = acc[h] + bias
            rating = 1.0 / (1.0 + jnp.exp(-logits))
            out_v[pl.ds(g * 2 * L + h * L, L)] = rating
        return 0

    lax.fori_loop(0, GROUPS // 2, group, 0)
    pltpu.sync_copy(out_v, out_hbm.at[pl.ds(base, BPW)])


@jax.jit
def _gmf(uidx, iidx, utab_t, itab_t, w, bias):
    mesh = plsc.VectorSubcoreMesh(core_axis_name="c", subcore_axis_name="s")
    kfn = pl.kernel(
        _gmf_body,
        out_type=jax.ShapeDtypeStruct((B,), jnp.float32),
        mesh=mesh,
        compiler_params=pltpu.CompilerParams(
            needs_layout_passes=False, use_tc_tiling_on_sc=True),
        scratch_types=[
            pltpu.VMEM((BPW,), jnp.int32),
            pltpu.VMEM((BPW,), jnp.int32),
            pltpu.VMEM((3, SLOTS, D, 128), jnp.float32),
            pltpu.VMEM((3, SLOTS, D, 128), jnp.float32),
            pltpu.VMEM((D,), jnp.float32),
            pltpu.VMEM((L,), jnp.float32),
            pltpu.VMEM((BPW,), jnp.float32),
            pltpu.SemaphoreType.DMA,
            pltpu.SemaphoreType.DMA,
        ],
    )
    return kfn(uidx, iidx, utab_t, itab_t, w, bias)


def kernel(user_indices, item_indices, user_table, item_table, W, b):
    uidx = user_indices.astype(jnp.int32)
    iidx = item_indices.astype(jnp.int32)
    w = W.astype(jnp.float32).reshape(D)
    bias = jnp.pad(b.astype(jnp.float32).reshape(1), (0, L - 1))
    out = _gmf(uidx, iidx, user_table.T, item_table.T, w, bias)
    return out.reshape(B, 1)
